# baseline (speedup 1.0000x reference)
import functools

import jax
import jax.numpy as jnp
from jax import lax
from jax.experimental import pallas as pl
from jax.experimental.pallas import tpu as pltpu
from jax.experimental.pallas import tpu_sc as plsc

_NC = 2
_NS = 16
_NW = _NC * _NS
_L = 16
_CHUNK = 512
_CROWS = _CHUNK // 4

_LN2 = 0.6931471805599453
_SQRT2 = 1.4142135623730951
_LOG_C = (
    -3.332947384568352e-06,
    0.9999100019104871,
    -0.49933572632078504,
    0.3376105578963719,
    -0.27109935070790736,
    0.17028616221812656,
)

_GATHER_DNUMS = lax.GatherDimensionNumbers(
    offset_dims=(), collapsed_slice_dims=(0,), start_index_map=(0,)
)


def _lane_gather(x, idx):
    return lax.gather(
        x,
        idx.reshape(_L, 1),
        _GATHER_DNUMS,
        (1,),
        mode=lax.GatherScatterMode.PROMISE_IN_BOUNDS,
    )


def _fast_log(x):
    xi = lax.bitcast_convert_type(x, jnp.int32)
    e = lax.shift_right_arithmetic(xi, 23) - 127
    m = lax.bitcast_convert_type(
        (xi & 0x007FFFFF) | 0x3F800000, jnp.float32
    )
    big = m > _SQRT2
    m = jnp.where(big, m * 0.5, m)
    ef = e.astype(jnp.float32) + jnp.where(big, 1.0, 0.0)
    z = m - 1.0
    p = jnp.float32(_LOG_C[5])
    for k in (4, 3, 2, 1, 0):
        p = p * z + jnp.float32(_LOG_C[k])
    return ef * jnp.float32(_LN2) + p


def _body(e1_hbm, e2_hbm, out_hbm, b1, b2, ob):
    wid = lax.axis_index("s") * _NC + lax.axis_index("c")
    n_rows = e1_hbm.shape[0]
    rows_w = n_rows // _NW
    n_chunks = rows_w // _CROWS

    iota = lax.iota(jnp.int32, _L)
    low2 = iota & 3
    high2 = iota - low2
    idx_a = [high2 + s1 for s1 in range(4)]
    idx_b = [low2 + 4 * s1 for s1 in range(4)]

    def one_elem(r, c0):
        p10 = jnp.exp(b1[r, pl.ds(c0, _L)])
        p11 = jnp.exp(b1[r, pl.ds(c0 + 16, _L)])
        p20 = jnp.exp(b2[r, pl.ds(c0, _L)])
        p21 = jnp.exp(b2[r, pl.ds(c0 + 16, _L)])
        acc0 = jnp.zeros((_L,), jnp.float32)
        acc1 = jnp.zeros((_L,), jnp.float32)
        for s1 in range(4):
            a0 = _lane_gather(p10, idx_a[s1])
            a1 = _lane_gather(p11, idx_a[s1])
            c0v = _lane_gather(p20, idx_b[s1])
            c1v = _lane_gather(p21, idx_b[s1])
            acc0 = acc0 + a0 * c0v + a1 * c1v
            acc1 = acc1 + a1 * c0v + a0 * c1v
        ob[r, pl.ds(c0, _L)] = _fast_log(acc0)
        ob[r, pl.ds(c0 + 16, _L)] = _fast_log(acc1)

    def chunk_body(ci, _):
        row0 = wid * rows_w + ci * _CROWS
        pltpu.sync_copy(e1_hbm.at[pl.ds(row0, _CROWS)], b1)
        pltpu.sync_copy(e2_hbm.at[pl.ds(row0, _CROWS)], b2)

        def elem_body(j, _):
            for k in range(4):
                one_elem(j, 32 * k)
            return ()

        lax.fori_loop(0, _CHUNK // 4, elem_body, ())
        pltpu.sync_copy(ob, out_hbm.at[pl.ds(row0, _CROWS)])
        return ()

    lax.fori_loop(0, n_chunks, chunk_body, ())


def kernel(e1, e2):
    shape5 = e1.shape
    n_rows = (shape5[0] * shape5[1] * 32) // 128
    e1f = e1.reshape(n_rows, 128)
    e2f = e2.reshape(n_rows, 128)
    mesh = plsc.VectorSubcoreMesh(core_axis_name="c", subcore_axis_name="s")
    run = pl.kernel(
        _body,
        out_type=jax.ShapeDtypeStruct((n_rows, 128), jnp.float32),
        mesh=mesh,
        scratch_types=[
            pltpu.VMEM((_CROWS, 128), jnp.float32),
            pltpu.VMEM((_CROWS, 128), jnp.float32),
            pltpu.VMEM((_CROWS, 128), jnp.float32),
        ],
        compiler_params=pltpu.CompilerParams(use_tc_tiling_on_sc=False),
    )
    out = run(e1f, e2f)
    return out.reshape(shape5)


# speedup vs baseline: 32.8648x; 32.8648x over previous
import functools

import jax
import jax.numpy as jnp
from jax import lax
from jax.experimental import pallas as pl
from jax.experimental.pallas import tpu as pltpu
from jax.experimental.pallas import tpu_sc as plsc

_NC = 2
_NS = 16
_NW = _NC * _NS
_L = 16
_G = 8

_LN2 = 0.6931471805599453
_SQRT2 = 1.4142135623730951
_LOG_C = (
    -3.332947384568352e-06,
    0.9999100019104871,
    -0.49933572632078504,
    0.3376105578963719,
    -0.27109935070790736,
    0.17028616221812656,
)


def _fast_log(x):
    xi = lax.bitcast_convert_type(x, jnp.int32)
    e = lax.shift_right_arithmetic(xi, 23) - 127
    m = lax.bitcast_convert_type(
        (xi & 0x007FFFFF) | 0x3F800000, jnp.float32
    )
    big = m > _SQRT2
    m = jnp.where(big, m * 0.5, m)
    ef = e.astype(jnp.float32) + jnp.where(big, 1.0, 0.0)
    z = m - 1.0
    p = jnp.float32(_LOG_C[5])
    for k in (4, 3, 2, 1, 0):
        p = p * z + jnp.float32(_LOG_C[k])
    return ef * jnp.float32(_LN2) + p


def _body(e1_hbm, e2_hbm, out_hbm, b1, b2, ob):
    wid = lax.axis_index("s") * _NC + lax.axis_index("c")
    n_slabs = e1_hbm.shape[0] // 8
    n_chunks = n_slabs // _G

    def block_body(t, _):
        g = t >> 3
        lb = (t & 7) * _L
        p1 = [
            [
                [jnp.exp(b1[(2 * g + u) * 4 + b, s1, pl.ds(lb, _L)])
                 for s1 in range(4)]
                for b in range(4)
            ]
            for u in range(2)
        ]
        for c in range(4):
            p2 = [
                [jnp.exp(b2[(2 * g + u2) * 4 + s1, c, pl.ds(lb, _L)])
                 for s1 in range(4)]
                for u2 in range(2)
            ]
            for b in range(4):
                acc0 = p1[0][b][0] * p2[0][0] + p1[1][b][0] * p2[1][0]
                acc1 = p1[1][b][0] * p2[0][0] + p1[0][b][0] * p2[1][0]
                for s1 in range(1, 4):
                    acc0 = acc0 + p1[0][b][s1] * p2[0][s1]
                    acc0 = acc0 + p1[1][b][s1] * p2[1][s1]
                    acc1 = acc1 + p1[1][b][s1] * p2[0][s1]
                    acc1 = acc1 + p1[0][b][s1] * p2[1][s1]
                ob[(2 * g + 0) * 4 + b, c, pl.ds(lb, _L)] = _fast_log(acc0)
                ob[(2 * g + 1) * 4 + b, c, pl.ds(lb, _L)] = _fast_log(acc1)
        return ()

    def chunk_body(ci, _):
        p0 = ci * 8 * _G
        pltpu.sync_copy(e1_hbm.at[pl.ds(p0, 8 * _G), wid], b1)
        pltpu.sync_copy(e2_hbm.at[pl.ds(p0, 8 * _G), wid], b2)
        lax.fori_loop(0, _G * 8, block_body, ())
        pltpu.sync_copy(ob, out_hbm.at[pl.ds(p0, 8 * _G), wid])
        return ()

    lax.fori_loop(0, n_chunks, chunk_body, ())


def kernel(e1, e2):
    b0, nb = e1.shape[0], e1.shape[1]
    tc = nb // 128
    rows = b0 * 2 * 4

    def to_view(x):
        x6 = x.reshape(b0, tc, 128, 2, 4, 4)
        x6 = jnp.transpose(x6, (0, 3, 4, 1, 5, 2))
        return x6.reshape(rows, tc, 4, 128)

    e1_v = to_view(e1)
    e2_v = to_view(e2)
    mesh = plsc.VectorSubcoreMesh(core_axis_name="c", subcore_axis_name="s")
    run = pl.kernel(
        _body,
        out_type=jax.ShapeDtypeStruct((rows, tc, 4, 128), jnp.float32),
        mesh=mesh,
        scratch_types=[
            pltpu.VMEM((8 * _G, 4, 128), jnp.float32),
            pltpu.VMEM((8 * _G, 4, 128), jnp.float32),
            pltpu.VMEM((8 * _G, 4, 128), jnp.float32),
        ],
        compiler_params=pltpu.CompilerParams(use_tc_tiling_on_sc=True),
    )
    out_v = run(e1_v, e2_v)
    out6 = out_v.reshape(b0, 2, 4, tc, 4, 128)
    out6 = jnp.transpose(out6, (0, 3, 5, 1, 2, 4))
    return out6.reshape(b0, nb, 2, 4, 4)


# speedup vs baseline: 39.0436x; 1.1880x over previous
import functools

import jax
import jax.numpy as jnp
from jax import lax
from jax.experimental import pallas as pl
from jax.experimental.pallas import tpu as pltpu
from jax.experimental.pallas import tpu_sc as plsc

_NC = 2
_NS = 16
_NW = _NC * _NS
_L = 16
_G = 8

_LN2 = 0.6931471805599453
_LOG_C = (
    0.00020421341095644419 - 127.0,
    1.4360975423091455,
    -0.6695137042412357,
    0.3122127797929613,
    -0.0791499317234476,
)


def _fast_log(x):
    xi = lax.bitcast_convert_type(x, jnp.int32)
    eb = lax.shift_right_arithmetic(xi, 23)
    m = lax.bitcast_convert_type(
        (xi & 0x007FFFFF) | 0x3F800000, jnp.float32
    )
    z = m - 1.0
    z2 = z * z
    a = jnp.float32(_LOG_C[0]) + jnp.float32(_LOG_C[1]) * z
    b = jnp.float32(_LOG_C[2]) + jnp.float32(_LOG_C[3]) * z
    p = a + z2 * (b + z2 * jnp.float32(_LOG_C[4]))
    return (eb.astype(jnp.float32) + p) * jnp.float32(_LN2)


def _body(e1_hbm, e2_hbm, out_hbm, b1, b2, ob):
    wid = lax.axis_index("s") * _NC + lax.axis_index("c")
    n_slabs = e1_hbm.shape[0] // 8
    n_chunks = n_slabs // _G

    def block_body(t, _):
        g = t >> 3
        lb = (t & 7) * _L
        pp = [[None] * 4 for _ in range(4)]
        pm = [[None] * 4 for _ in range(4)]
        for b in range(4):
            for s1 in range(4):
                x0 = jnp.exp(b1[(2 * g + 0) * 4 + b, s1, pl.ds(lb, _L)])
                x1 = jnp.exp(b1[(2 * g + 1) * 4 + b, s1, pl.ds(lb, _L)])
                pp[b][s1] = x0 + x1
                pm[b][s1] = x0 - x1
        for c in range(4):
            qp = [None] * 4
            qm = [None] * 4
            for s1 in range(4):
                y0 = jnp.exp(b2[(2 * g + 0) * 4 + s1, c, pl.ds(lb, _L)])
                y1 = jnp.exp(b2[(2 * g + 1) * 4 + s1, c, pl.ds(lb, _L)])
                qp[s1] = (y0 + y1) * 0.5
                qm[s1] = (y0 - y1) * 0.5
            for b in range(4):
                s = (pp[b][0] * qp[0] + pp[b][1] * qp[1]) + (
                    pp[b][2] * qp[2] + pp[b][3] * qp[3]
                )
                d = (pm[b][0] * qm[0] + pm[b][1] * qm[1]) + (
                    pm[b][2] * qm[2] + pm[b][3] * qm[3]
                )
                ob[(2 * g + 0) * 4 + b, c, pl.ds(lb, _L)] = _fast_log(s + d)
                ob[(2 * g + 1) * 4 + b, c, pl.ds(lb, _L)] = _fast_log(s - d)
        return ()

    def chunk_body(ci, _):
        p0 = ci * 8 * _G
        pltpu.sync_copy(e1_hbm.at[pl.ds(p0, 8 * _G), wid], b1)
        pltpu.sync_copy(e2_hbm.at[pl.ds(p0, 8 * _G), wid], b2)
        lax.fori_loop(0, _G * 8, block_body, ())
        pltpu.sync_copy(ob, out_hbm.at[pl.ds(p0, 8 * _G), wid])
        return ()

    lax.fori_loop(0, n_chunks, chunk_body, ())


def kernel(e1, e2):
    b0, nb = e1.shape[0], e1.shape[1]
    tc = nb // 128
    rows = b0 * 2 * 4

    def to_view(x):
        x6 = x.reshape(b0, tc, 128, 2, 4, 4)
        x6 = jnp.transpose(x6, (0, 3, 4, 1, 5, 2))
        return x6.reshape(rows, tc, 4, 128)

    e1_v = to_view(e1)
    e2_v = to_view(e2)
    mesh = plsc.VectorSubcoreMesh(core_axis_name="c", subcore_axis_name="s")
    run = pl.kernel(
        _body,
        out_type=jax.ShapeDtypeStruct((rows, tc, 4, 128), jnp.float32),
        mesh=mesh,
        scratch_types=[
            pltpu.VMEM((8 * _G, 4, 128), jnp.float32),
            pltpu.VMEM((8 * _G, 4, 128), jnp.float32),
            pltpu.VMEM((8 * _G, 4, 128), jnp.float32),
        ],
        compiler_params=pltpu.CompilerParams(use_tc_tiling_on_sc=True),
    )
    out_v = run(e1_v, e2_v)
    out6 = out_v.reshape(b0, 2, 4, tc, 4, 128)
    out6 = jnp.transpose(out6, (0, 3, 5, 1, 2, 4))
    return out6.reshape(b0, nb, 2, 4, 4)


# speedup vs baseline: 48.8179x; 1.2503x over previous
import functools

import jax
import jax.numpy as jnp
from jax import lax
from jax.experimental import pallas as pl
from jax.experimental.pallas import tpu as pltpu
from jax.experimental.pallas import tpu_sc as plsc

_NC = 2
_NS = 16
_NW = _NC * _NS
_L = 16
_G = 4

_LN2 = 0.6931471805599453
_LOG_C = (
    0.00020421341095644419 - 127.0,
    1.4360975423091455,
    -0.6695137042412357,
    0.3122127797929613,
    -0.0791499317234476,
)


def _fast_log(x):
    xi = lax.bitcast_convert_type(x, jnp.int32)
    eb = lax.shift_right_arithmetic(xi, 23)
    m = lax.bitcast_convert_type(
        (xi & 0x007FFFFF) | 0x3F800000, jnp.float32
    )
    z = m - 1.0
    z2 = z * z
    a = jnp.float32(_LOG_C[0]) + jnp.float32(_LOG_C[1]) * z
    b = jnp.float32(_LOG_C[2]) + jnp.float32(_LOG_C[3]) * z
    p = a + z2 * (b + z2 * jnp.float32(_LOG_C[4]))
    return (eb.astype(jnp.float32) + p) * jnp.float32(_LN2)


def _body(e1_hbm, e2_hbm, out_hbm, b1s, b2s, obs, sin1, sin2, sout):
    wid = lax.axis_index("s") * _NC + lax.axis_index("c")
    n_slabs = e1_hbm.shape[0] // 8
    n_chunks = n_slabs // _G

    def make_block_body(b1, b2, ob):
      def block_body(t, _):
        g = t >> 3
        lb = (t & 7) * _L
        pp = [[None] * 4 for _ in range(4)]
        pm = [[None] * 4 for _ in range(4)]
        for b in range(4):
            for s1 in range(4):
                x0 = jnp.exp(b1[(2 * g + 0) * 4 + b, s1, pl.ds(lb, _L)])
                x1 = jnp.exp(b1[(2 * g + 1) * 4 + b, s1, pl.ds(lb, _L)])
                pp[b][s1] = x0 + x1
                pm[b][s1] = x0 - x1
        for c in range(4):
            qp = [None] * 4
            qm = [None] * 4
            for s1 in range(4):
                y0 = jnp.exp(b2[(2 * g + 0) * 4 + s1, c, pl.ds(lb, _L)])
                y1 = jnp.exp(b2[(2 * g + 1) * 4 + s1, c, pl.ds(lb, _L)])
                qp[s1] = (y0 + y1) * 0.5
                qm[s1] = (y0 - y1) * 0.5
            for b in range(4):
                s = (pp[b][0] * qp[0] + pp[b][1] * qp[1]) + (
                    pp[b][2] * qp[2] + pp[b][3] * qp[3]
                )
                d = (pm[b][0] * qm[0] + pm[b][1] * qm[1]) + (
                    pm[b][2] * qm[2] + pm[b][3] * qm[3]
                )
                ob[(2 * g + 0) * 4 + b, c, pl.ds(lb, _L)] = _fast_log(s + d)
                ob[(2 * g + 1) * 4 + b, c, pl.ds(lb, _L)] = _fast_log(s - d)
        return ()
      return block_body

    def in_copies(ci, par):
        p0 = ci * 8 * _G
        return (
            pltpu.make_async_copy(
                e1_hbm.at[pl.ds(p0, 8 * _G), wid], b1s[par], sin1[par]
            ),
            pltpu.make_async_copy(
                e2_hbm.at[pl.ds(p0, 8 * _G), wid], b2s[par], sin2[par]
            ),
        )

    def out_copy(ci, par):
        p0 = ci * 8 * _G
        return pltpu.make_async_copy(
            obs[par], out_hbm.at[pl.ds(p0, 8 * _G), wid], sout[par]
        )

    n_pairs = n_chunks // 2

    for cp in in_copies(0, 0):
        cp.start()

    def pair_body(i, _):
        for par in (0, 1):
            ci = 2 * i + par
            if par == 0:
                for cp in in_copies(ci + 1, 1):
                    cp.start()
            else:
                @pl.when(i < n_pairs - 1)
                def _():
                    for cp in in_copies(ci + 1, 0):
                        cp.start()
            for cp in in_copies(ci, par):
                cp.wait()

            @pl.when(i > 0)
            def _():
                out_copy(ci, par).wait()

            lax.fori_loop(
                0, _G * 8, make_block_body(b1s[par], b2s[par], obs[par]), ()
            )
            out_copy(ci, par).start()
        return ()

    lax.fori_loop(0, n_pairs, pair_body, ())
    out_copy(n_chunks - 2, 0).wait()
    out_copy(n_chunks - 1, 1).wait()


def kernel(e1, e2):
    b0, nb = e1.shape[0], e1.shape[1]
    tc = nb // 128
    rows = b0 * 2 * 4

    def to_view(x):
        x6 = x.reshape(b0, tc, 128, 2, 4, 4)
        x6 = jnp.transpose(x6, (0, 3, 4, 1, 5, 2))
        return x6.reshape(rows, tc, 4, 128)

    e1_v = to_view(e1)
    e2_v = to_view(e2)
    mesh = plsc.VectorSubcoreMesh(core_axis_name="c", subcore_axis_name="s")
    run = pl.kernel(
        _body,
        out_type=jax.ShapeDtypeStruct((rows, tc, 4, 128), jnp.float32),
        mesh=mesh,
        scratch_types=[
            (pltpu.VMEM((8 * _G, 4, 128), jnp.float32),) * 2,
            (pltpu.VMEM((8 * _G, 4, 128), jnp.float32),) * 2,
            (pltpu.VMEM((8 * _G, 4, 128), jnp.float32),) * 2,
            (pltpu.SemaphoreType.DMA,) * 2,
            (pltpu.SemaphoreType.DMA,) * 2,
            (pltpu.SemaphoreType.DMA,) * 2,
        ],
        compiler_params=pltpu.CompilerParams(use_tc_tiling_on_sc=True),
    )
    out_v = run(e1_v, e2_v)
    out6 = out_v.reshape(b0, 2, 4, tc, 4, 128)
    out6 = jnp.transpose(out6, (0, 3, 5, 1, 2, 4))
    return out6.reshape(b0, nb, 2, 4, 4)


# speedup vs baseline: 51.1656x; 1.0481x over previous
import functools

import jax
import jax.numpy as jnp
from jax import lax
from jax.experimental import pallas as pl
from jax.experimental.pallas import tpu as pltpu
from jax.experimental.pallas import tpu_sc as plsc

_NC = 2
_NS = 16
_NW = _NC * _NS
_L = 16
_G = 4

_LN2 = 0.6931471805599453
_LOG_C = (
    0.0013347571220687637 - 127.0,
    1.413484124102092,
    -0.5677503543107336,
    0.15391291634606508,
)


def _fast_log(x):
    xi = lax.bitcast_convert_type(x, jnp.int32)
    eb = lax.shift_right_arithmetic(xi, 23)
    m = lax.bitcast_convert_type(
        (xi & 0x007FFFFF) | 0x3F800000, jnp.float32
    )
    z = m - 1.0
    z2 = z * z
    a = jnp.float32(_LOG_C[0]) + jnp.float32(_LOG_C[1]) * z
    p = a + z2 * (jnp.float32(_LOG_C[2]) + jnp.float32(_LOG_C[3]) * z)
    return (eb.astype(jnp.float32) + p) * jnp.float32(_LN2)


def _body(e1_hbm, e2_hbm, out_hbm, b1s, b2s, obs, sin1, sin2, sout):
    wid = lax.axis_index("s") * _NC + lax.axis_index("c")
    n_slabs = e1_hbm.shape[0] // 8
    n_chunks = n_slabs // _G

    def make_block_body(b1, b2, ob):
      def block_body(t, _):
        g = t >> 3
        lb = (t & 7) * _L
        pp = [[None] * 4 for _ in range(4)]
        pm = [[None] * 4 for _ in range(4)]
        for b in range(4):
            for s1 in range(4):
                x0 = jnp.exp(b1[(2 * g + 0) * 4 + b, s1, pl.ds(lb, _L)])
                x1 = jnp.exp(b1[(2 * g + 1) * 4 + b, s1, pl.ds(lb, _L)])
                pp[b][s1] = x0 + x1
                pm[b][s1] = x0 - x1
        for c in range(4):
            qp = [None] * 4
            qm = [None] * 4
            for s1 in range(4):
                y0 = jnp.exp(b2[(2 * g + 0) * 4 + s1, c, pl.ds(lb, _L)])
                y1 = jnp.exp(b2[(2 * g + 1) * 4 + s1, c, pl.ds(lb, _L)])
                qp[s1] = (y0 + y1) * 0.5
                qm[s1] = (y0 - y1) * 0.5
            for b in range(4):
                s = (pp[b][0] * qp[0] + pp[b][1] * qp[1]) + (
                    pp[b][2] * qp[2] + pp[b][3] * qp[3]
                )
                d = (pm[b][0] * qm[0] + pm[b][1] * qm[1]) + (
                    pm[b][2] * qm[2] + pm[b][3] * qm[3]
                )
                ob[(2 * g + 0) * 4 + b, c, pl.ds(lb, _L)] = _fast_log(s + d)
                ob[(2 * g + 1) * 4 + b, c, pl.ds(lb, _L)] = _fast_log(s - d)
        return ()
      return block_body

    def in_copies(ci, par):
        p0 = ci * 8 * _G
        return (
            pltpu.make_async_copy(
                e1_hbm.at[pl.ds(p0, 8 * _G), wid], b1s[par], sin1[par]
            ),
            pltpu.make_async_copy(
                e2_hbm.at[pl.ds(p0, 8 * _G), wid], b2s[par], sin2[par]
            ),
        )

    def out_copy(ci, par):
        p0 = ci * 8 * _G
        return pltpu.make_async_copy(
            obs[par], out_hbm.at[pl.ds(p0, 8 * _G), wid], sout[par]
        )

    n_pairs = n_chunks // 2

    for cp in in_copies(0, 0):
        cp.start()

    def pair_body(i, _):
        for par in (0, 1):
            ci = 2 * i + par
            if par == 0:
                for cp in in_copies(ci + 1, 1):
                    cp.start()
            else:
                @pl.when(i < n_pairs - 1)
                def _():
                    for cp in in_copies(ci + 1, 0):
                        cp.start()
            for cp in in_copies(ci, par):
                cp.wait()

            @pl.when(i > 0)
            def _():
                out_copy(ci, par).wait()

            lax.fori_loop(
                0, _G * 8, make_block_body(b1s[par], b2s[par], obs[par]), ()
            )
            out_copy(ci, par).start()
        return ()

    lax.fori_loop(0, n_pairs, pair_body, ())
    out_copy(n_chunks - 2, 0).wait()
    out_copy(n_chunks - 1, 1).wait()


def kernel(e1, e2):
    b0, nb = e1.shape[0], e1.shape[1]
    tc = nb // 128
    rows = b0 * 2 * 4

    def to_view(x):
        x6 = x.reshape(b0, tc, 128, 2, 4, 4)
        x6 = jnp.transpose(x6, (0, 3, 4, 1, 5, 2))
        return x6.reshape(rows, tc, 4, 128)

    e1_v = to_view(e1)
    e2_v = to_view(e2)
    mesh = plsc.VectorSubcoreMesh(core_axis_name="c", subcore_axis_name="s")
    run = pl.kernel(
        _body,
        out_type=jax.ShapeDtypeStruct((rows, tc, 4, 128), jnp.float32),
        mesh=mesh,
        scratch_types=[
            (pltpu.VMEM((8 * _G, 4, 128), jnp.float32),) * 2,
            (pltpu.VMEM((8 * _G, 4, 128), jnp.float32),) * 2,
            (pltpu.VMEM((8 * _G, 4, 128), jnp.float32),) * 2,
            (pltpu.SemaphoreType.DMA,) * 2,
            (pltpu.SemaphoreType.DMA,) * 2,
            (pltpu.SemaphoreType.DMA,) * 2,
        ],
        compiler_params=pltpu.CompilerParams(use_tc_tiling_on_sc=True),
    )
    out_v = run(e1_v, e2_v)
    out6 = out_v.reshape(b0, 2, 4, tc, 4, 128)
    out6 = jnp.transpose(out6, (0, 3, 5, 1, 2, 4))
    return out6.reshape(b0, nb, 2, 4, 4)


# speedup vs baseline: 52.7523x; 1.0310x over previous
import functools

import jax
import jax.numpy as jnp
from jax import lax
from jax.experimental import pallas as pl
from jax.experimental.pallas import tpu as pltpu
from jax.experimental.pallas import tpu_sc as plsc

_NC = 2
_NS = 16
_NW = _NC * _NS
_L = 16
_G = 4

_LN2 = 0.6931471805599453
_LOG_C = (
    0.0013347571220687637 - 128.0,
    1.413484124102092,
    -0.5677503543107336,
    0.15391291634606508,
)


def _fast_log(x):
    xi = lax.bitcast_convert_type(x, jnp.int32)
    eb = lax.shift_right_arithmetic(xi, 23)
    m = lax.bitcast_convert_type(
        (xi & 0x007FFFFF) | 0x3F800000, jnp.float32
    )
    z = m - 1.0
    z2 = z * z
    a = jnp.float32(_LOG_C[0]) + jnp.float32(_LOG_C[1]) * z
    p = a + z2 * (jnp.float32(_LOG_C[2]) + jnp.float32(_LOG_C[3]) * z)
    return (eb.astype(jnp.float32) + p) * jnp.float32(_LN2)


def _body(e1_hbm, e2_hbm, out_hbm, b1s, b2s, obs, sin1, sin2, sout):
    wid = lax.axis_index("s") * _NC + lax.axis_index("c")
    n_slabs = e1_hbm.shape[0] // 8
    n_chunks = n_slabs // _G

    def make_block_body(b1, b2, ob):
      def block_body(t, _):
        g = t >> 3
        lb = (t & 7) * _L
        pp = [[None] * 4 for _ in range(4)]
        pm = [[None] * 4 for _ in range(4)]
        for b in range(4):
            for s1 in range(4):
                x0 = jnp.exp(b1[(2 * g + 0) * 4 + b, s1, pl.ds(lb, _L)])
                x1 = jnp.exp(b1[(2 * g + 1) * 4 + b, s1, pl.ds(lb, _L)])
                pp[b][s1] = x0 + x1
                pm[b][s1] = x0 - x1
        for c in range(4):
            qp = [None] * 4
            qm = [None] * 4
            for s1 in range(4):
                y0 = jnp.exp(b2[(2 * g + 0) * 4 + s1, c, pl.ds(lb, _L)])
                y1 = jnp.exp(b2[(2 * g + 1) * 4 + s1, c, pl.ds(lb, _L)])
                qp[s1] = y0 + y1
                qm[s1] = y0 - y1
            for b in range(4):
                s = (pp[b][0] * qp[0] + pp[b][1] * qp[1]) + (
                    pp[b][2] * qp[2] + pp[b][3] * qp[3]
                )
                d = (pm[b][0] * qm[0] + pm[b][1] * qm[1]) + (
                    pm[b][2] * qm[2] + pm[b][3] * qm[3]
                )
                ob[(2 * g + 0) * 4 + b, c, pl.ds(lb, _L)] = _fast_log(s + d)
                ob[(2 * g + 1) * 4 + b, c, pl.ds(lb, _L)] = _fast_log(s - d)
        return ()
      return block_body

    def in_copies(ci, par):
        p0 = ci * 8 * _G
        return (
            pltpu.make_async_copy(
                e1_hbm.at[pl.ds(p0, 8 * _G), wid], b1s[par], sin1[par]
            ),
            pltpu.make_async_copy(
                e2_hbm.at[pl.ds(p0, 8 * _G), wid], b2s[par], sin2[par]
            ),
        )

    def out_copy(ci, par):
        p0 = ci * 8 * _G
        return pltpu.make_async_copy(
            obs[par], out_hbm.at[pl.ds(p0, 8 * _G), wid], sout[par]
        )

    n_pairs = n_chunks // 2

    for cp in in_copies(0, 0):
        cp.start()

    def pair_body(i, _):
        for par in (0, 1):
            ci = 2 * i + par
            if par == 0:
                for cp in in_copies(ci + 1, 1):
                    cp.start()
            else:
                @pl.when(i < n_pairs - 1)
                def _():
                    for cp in in_copies(ci + 1, 0):
                        cp.start()
            for cp in in_copies(ci, par):
                cp.wait()

            @pl.when(i > 0)
            def _():
                out_copy(ci, par).wait()

            lax.fori_loop(
                0, _G * 8, make_block_body(b1s[par], b2s[par], obs[par]), ()
            )
            out_copy(ci, par).start()
        return ()

    lax.fori_loop(0, n_pairs, pair_body, ())
    out_copy(n_chunks - 2, 0).wait()
    out_copy(n_chunks - 1, 1).wait()


def kernel(e1, e2):
    b0, nb = e1.shape[0], e1.shape[1]
    tc = nb // 128
    rows = b0 * 2 * 4

    def to_view(x):
        x6 = x.reshape(b0, tc, 128, 2, 4, 4)
        x6 = jnp.transpose(x6, (0, 3, 4, 1, 5, 2))
        return x6.reshape(rows, tc, 4, 128)

    e1_v = to_view(e1)
    e2_v = to_view(e2)
    mesh = plsc.VectorSubcoreMesh(core_axis_name="c", subcore_axis_name="s")
    run = pl.kernel(
        _body,
        out_type=jax.ShapeDtypeStruct((rows, tc, 4, 128), jnp.float32),
        mesh=mesh,
        scratch_types=[
            (pltpu.VMEM((8 * _G, 4, 128), jnp.float32),) * 2,
            (pltpu.VMEM((8 * _G, 4, 128), jnp.float32),) * 2,
            (pltpu.VMEM((8 * _G, 4, 128), jnp.float32),) * 2,
            (pltpu.SemaphoreType.DMA,) * 2,
            (pltpu.SemaphoreType.DMA,) * 2,
            (pltpu.SemaphoreType.DMA,) * 2,
        ],
        compiler_params=pltpu.CompilerParams(use_tc_tiling_on_sc=True),
    )
    out_v = run(e1_v, e2_v)
    out6 = out_v.reshape(b0, 2, 4, tc, 4, 128)
    out6 = jnp.transpose(out6, (0, 3, 5, 1, 2, 4))
    return out6.reshape(b0, nb, 2, 4, 4)
